# baseline (speedup 1.0000x reference)
import jax
import jax.numpy as jnp
from jax import lax
from jax.experimental import pallas as pl
from jax.experimental.pallas import tpu as pltpu
from jax.experimental.pallas import tpu_sc as plsc

N = 10000
NP = 10240
D = 128
E = 320000
NC = 2
NS = 16
L = 16
HALF = D // NC
EPT = E // NS
K = 128
NB = 4
NFULL = EPT // K
TAIL = EPT - NFULL * K
RPT = NP // NS
SUB = 64
NSUB = RPT // SUB

_f32 = jnp.float32
_i32 = jnp.int32


def _body(poi, tar_r, tar_c, tarv, src_r, src_c, srcv, out_c, a_h, b_h,
          acc,
          cv0, cv1, cv2, cv3, rv0, rv1, rv2, rv3,
          vv0, vv1, vv2, vv3, rb0, rb1, rb2, rb3,
          cbuf, tbuf, t2,
          sl0, sl1, sl2, sl3, sg0, sg1, sg2, sg3, ss0, ss1, ss2, ss3):
    colvs = [cv0, cv1, cv2, cv3]
    rowvs = [rv0, rv1, rv2, rv3]
    valvs = [vv0, vv1, vv2, vv3]
    rowbs = [rb0, rb1, rb2, rb3]
    semL = [sl0, sl1, sl2, sl3]
    semG = [sg0, sg1, sg2, sg3]
    semS = [ss0, ss1, ss2, ss3]

    c = lax.axis_index("c")
    s = lax.axis_index("s")
    cN = c * NP
    row0 = s * RPT
    ebase = s * EPT

    def zero_buf(buf):
        zero = jnp.zeros((L,), _f32)

        def bd(i, _):
            for k in range(HALF // L):
                buf[i, pl.ds(k * L, L)] = zero
            return 0

        lax.fori_loop(0, buf.shape[0], bd, 0)

    def zero_acc():
        zero_buf(tbuf)
        for sub in range(NSUB):
            pltpu.sync_copy(tbuf, acc.at[pl.ds(row0 + sub * SUB, SUB)])

    def scale_chunk(b):
        vb, rb = valvs[b], rowbs[b]

        def gb(g, _):
            valvec = vb[pl.ds(g * L, L)]
            for j in range(L):
                e = g * L + j
                v = valvec[j]
                loads = [rb[e, pl.ds(k * L, L)] for k in range(HALF // L)]
                prods = [x * v for x in loads]
                for k in range(HALF // L):
                    rb[e, pl.ds(k * L, L)] = prods[k]
            return 0

        lax.fori_loop(0, K // L, gb, 0)

    def adjust_cols(b):
        cb_ = colvs[b]
        off = jnp.full((L,), 0, _i32) + cN

        def ab(g, _):
            cb_[pl.ds(g * L, L)] = cb_[pl.ds(g * L, L)] + off
            return 0

        lax.fori_loop(0, K // L, ab, 0)

    def spmm(row_ref, col_ref, val_ref, x_ref):
        def issue_loads(ci, b):
            eb = ebase + ci * K
            pltpu.async_copy(col_ref.at[pl.ds(eb, K)], colvs[b], semL[b])
            pltpu.async_copy(row_ref.at[pl.ds(eb, K)], rowvs[b], semL[b])
            pltpu.async_copy(val_ref.at[pl.ds(eb, K)], valvs[b], semL[b])

        def wait_loads(b):
            pltpu.make_async_copy(col_ref.at[pl.ds(0, K)], colvs[b], semL[b]).wait()
            pltpu.make_async_copy(row_ref.at[pl.ds(0, K)], rowvs[b], semL[b]).wait()
            pltpu.make_async_copy(val_ref.at[pl.ds(0, K)], valvs[b], semL[b]).wait()

        def issue_gather(b):
            pltpu.async_copy(x_ref.at[colvs[b]], rowbs[b], semG[b])

        def wait_gather(b):
            pltpu.make_async_copy(x_ref.at[colvs[b]], rowbs[b], semG[b]).wait()

        def issue_scatter(b):
            pltpu.async_copy(rowbs[b], acc.at[rowvs[b]], semS[b], add=True)

        def wait_scatter(b):
            pltpu.make_async_copy(rowbs[b], acc.at[rowvs[b]], semS[b]).wait()

        issue_loads(0, 0)
        issue_loads(1, 1)
        wait_loads(0)
        adjust_cols(0)
        issue_gather(0)

        def gb(gi, _):
            for b in range(NB):
                ci = gi * NB + b
                b1 = (b + 1) % NB
                b2 = (b + 2) % NB

                @pl.when(ci + 1 < NFULL)
                def _():
                    wait_loads(b1)
                    adjust_cols(b1)
                    issue_gather(b1)

                @pl.when(ci >= 2)
                def _():
                    wait_scatter(b2)

                @pl.when(ci + 2 < NFULL)
                def _():
                    issue_loads(ci + 2, b2)

                wait_gather(b)
                scale_chunk(b)
                issue_scatter(b)
            return 0

        lax.fori_loop(0, NFULL // NB, gb, 0)
        wait_scatter((NFULL - 2) % NB)
        wait_scatter((NFULL - 1) % NB)

        zi = jnp.zeros((L,), _i32)
        zf = jnp.zeros((L,), _f32)
        for g in range(K // L):
            cv0[pl.ds(g * L, L)] = zi
            rv0[pl.ds(g * L, L)] = zi
            vv0[pl.ds(g * L, L)] = zf
        eb = ebase + NFULL * K
        pltpu.sync_copy(col_ref.at[pl.ds(eb, TAIL)], cv0.at[pl.ds(0, TAIL)])
        pltpu.sync_copy(row_ref.at[pl.ds(eb, TAIL)], rv0.at[pl.ds(0, TAIL)])
        pltpu.sync_copy(val_ref.at[pl.ds(eb, TAIL)], vv0.at[pl.ds(0, TAIL)])
        adjust_cols(0)
        pltpu.async_copy(x_ref.at[cv0], rb0, semG[0]).wait()
        scale_chunk(0)
        pltpu.sync_copy(rb0, acc.at[rv0], add=True)

    def acc_to(dst_ref, rezero):
        for sub in range(NSUB):
            r0l = row0 + sub * SUB
            pltpu.sync_copy(acc.at[pl.ds(r0l, SUB)], tbuf)
            pltpu.sync_copy(tbuf, dst_ref.at[pl.ds(cN + r0l, SUB)])
        if rezero:
            zero_acc()

    def elementwise(x_src_ref, rezero):
        for sub in range(NSUB):
            r0l = row0 + sub * SUB
            r0g = cN + r0l
            pltpu.sync_copy(acc.at[pl.ds(r0l, SUB)], tbuf)
            pltpu.sync_copy(x_src_ref.at[pl.ds(r0g, SUB)], t2)

            def eb(i, _):
                ci = sub * SUB + i
                for k in range(HALF // L):
                    sl = pl.ds(k * L, L)
                    nv = tbuf[i, sl] + t2[i, sl]
                    tbuf[i, sl] = nv
                    cbuf[ci, sl] = cbuf[ci, sl] + nv
                return 0

            lax.fori_loop(0, SUB, eb, 0)
            pltpu.sync_copy(tbuf, a_h.at[pl.ds(r0g, SUB)])
        if rezero:
            zero_acc()

    pltpu.sync_copy(poi.at[pl.ds(cN + row0, RPT)], cbuf)
    zero_acc()
    plsc.subcore_barrier()

    for layer in range(2):
        x_src = poi if layer == 0 else a_h
        spmm(tar_r, tar_c, tarv, x_src)
        plsc.subcore_barrier()
        acc_to(b_h, rezero=True)
        plsc.subcore_barrier()
        spmm(src_r, src_c, srcv, b_h)
        plsc.subcore_barrier()
        elementwise(x_src, rezero=(layer == 0))
        plsc.subcore_barrier()

    third = jnp.float32(1.0 / 3.0)
    for sub in range(NSUB):
        def fb(i, _):
            for k in range(HALF // L):
                sl = pl.ds(k * L, L)
                tbuf[i, sl] = cbuf[sub * SUB + i, sl] * third
            return 0

        lax.fori_loop(0, SUB, fb, 0)
        pltpu.sync_copy(tbuf, out_c.at[pl.ds(cN + row0 + sub * SUB, SUB)])


@jax.jit
def kernel(poi_embs, src_indices, src_values, tar_indices, tar_values):
    poi_cat = poi_embs.reshape(N, NC, HALF).transpose(1, 0, 2)
    poi_cat = jnp.pad(poi_cat, ((0, 0), (0, NP - N), (0, 0))).reshape(NC * NP, HALF)

    mesh = plsc.VectorSubcoreMesh(
        core_axis_name="c", subcore_axis_name="s", num_cores=NC, num_subcores=NS
    )
    run = pl.kernel(
        _body,
        out_type=[
            jax.ShapeDtypeStruct((NC * NP, HALF), _f32),
            jax.ShapeDtypeStruct((NC * NP, HALF), _f32),
            jax.ShapeDtypeStruct((NC * NP, HALF), _f32),
        ],
        mesh=mesh,
        compiler_params=pltpu.CompilerParams(use_tc_tiling_on_sc=False),
        scratch_types=[pltpu.VMEM_SHARED((NP, HALF), _f32)]
        + [pltpu.VMEM((K,), _i32) for _ in range(2 * NB)]
        + [pltpu.VMEM((K,), _f32) for _ in range(NB)]
        + [pltpu.VMEM((K, HALF), _f32) for _ in range(NB)]
        + [
            pltpu.VMEM((RPT, HALF), _f32),
            pltpu.VMEM((SUB, HALF), _f32),
            pltpu.VMEM((SUB, HALF), _f32),
        ]
        + [pltpu.SemaphoreType.DMA for _ in range(3 * NB)],
    )
    out_c, _, _ = run(poi_cat, tar_indices[0], tar_indices[1], tar_values,
                      src_indices[0], src_indices[1], src_values)
    return out_c.reshape(NC, NP, HALF)[:, :N].transpose(1, 0, 2).reshape(N, D)


# speedup vs baseline: 1.1192x; 1.1192x over previous
import jax
import jax.numpy as jnp
from jax import lax
from jax.experimental import pallas as pl
from jax.experimental.pallas import tpu as pltpu
from jax.experimental.pallas import tpu_sc as plsc

N = 10000
NP = 10240
D = 128
E = 320000
NC = 2
NS = 16
L = 16
HALF = D // NC
EPT = E // NS
K = 128
NB = 4
NFULL = EPT // K
TAIL = EPT - NFULL * K
RPT = NP // NS
SUB = 64
NSUB = RPT // SUB

_f32 = jnp.float32
_i32 = jnp.int32


def _body(poi, tar_r, tar_c, tarv, src_r, src_c, srcv, out_c,
          acc, xa,
          cv0, cv1, cv2, cv3, rv0, rv1, rv2, rv3,
          vv0, vv1, vv2, vv3, rb0, rb1, rb2, rb3,
          tbuf, t2,
          sl0, sl1, sl2, sl3, sg0, sg1, sg2, sg3, ss0, ss1, ss2, ss3):
    colvs = [cv0, cv1, cv2, cv3]
    rowvs = [rv0, rv1, rv2, rv3]
    valvs = [vv0, vv1, vv2, vv3]
    rowbs = [rb0, rb1, rb2, rb3]
    semL = [sl0, sl1, sl2, sl3]
    semG = [sg0, sg1, sg2, sg3]
    semS = [ss0, ss1, ss2, ss3]

    c = lax.axis_index("c")
    s = lax.axis_index("s")
    cN = c * NP
    row0 = s * RPT
    ebase = s * EPT

    def zero_tbuf():
        zero = jnp.zeros((L,), _f32)

        def bd(i, _):
            for k in range(HALF // L):
                tbuf[i, pl.ds(k * L, L)] = zero
            return 0

        lax.fori_loop(0, SUB, bd, 0)

    def zero_acc_stripe():
        for sub in range(NSUB):
            pltpu.sync_copy(tbuf, acc.at[pl.ds(row0 + sub * SUB, SUB)])

    def scale_chunk(b):
        vb, rb = valvs[b], rowbs[b]

        def gb(g, _):
            valvec = vb[pl.ds(g * L, L)]
            for j in range(L):
                e = g * L + j
                v = valvec[j]
                loads = [rb[e, pl.ds(k * L, L)] for k in range(HALF // L)]
                prods = [x * v for x in loads]
                for k in range(HALF // L):
                    rb[e, pl.ds(k * L, L)] = prods[k]
            return 0

        lax.fori_loop(0, K // L, gb, 0)

    def spmm(row_ref, col_ref, val_ref, x_sp, dst):
        def issue_loads(ci, b):
            eb = ebase + ci * K
            pltpu.async_copy(col_ref.at[pl.ds(eb, K)], colvs[b], semL[b])
            pltpu.async_copy(row_ref.at[pl.ds(eb, K)], rowvs[b], semL[b])
            pltpu.async_copy(val_ref.at[pl.ds(eb, K)], valvs[b], semL[b])

        def wait_loads(b):
            pltpu.make_async_copy(col_ref.at[pl.ds(0, K)], colvs[b], semL[b]).wait()
            pltpu.make_async_copy(row_ref.at[pl.ds(0, K)], rowvs[b], semL[b]).wait()
            pltpu.make_async_copy(val_ref.at[pl.ds(0, K)], valvs[b], semL[b]).wait()

        def issue_gather(b):
            pltpu.async_copy(x_sp.at[colvs[b]], rowbs[b], semG[b])

        def wait_gather(b):
            pltpu.make_async_copy(x_sp.at[colvs[b]], rowbs[b], semG[b]).wait()

        def issue_scatter(b):
            pltpu.async_copy(rowbs[b], dst.at[rowvs[b]], semS[b], add=True)

        def wait_scatter(b):
            pltpu.make_async_copy(rowbs[b], dst.at[rowvs[b]], semS[b]).wait()

        issue_loads(0, 0)
        issue_loads(1, 1)
        wait_loads(0)
        issue_gather(0)

        def gb(gi, _):
            for b in range(NB):
                ci = gi * NB + b
                b1 = (b + 1) % NB
                b2 = (b + 2) % NB

                @pl.when(ci + 1 < NFULL)
                def _():
                    wait_loads(b1)
                    issue_gather(b1)

                @pl.when(ci >= 2)
                def _():
                    wait_scatter(b2)

                @pl.when(ci + 2 < NFULL)
                def _():
                    issue_loads(ci + 2, b2)

                wait_gather(b)
                scale_chunk(b)
                issue_scatter(b)
            return 0

        lax.fori_loop(0, NFULL // NB, gb, 0)
        wait_scatter((NFULL - 2) % NB)
        wait_scatter((NFULL - 1) % NB)

        zi = jnp.zeros((L,), _i32)
        zf = jnp.zeros((L,), _f32)
        for g in range(K // L):
            cv0[pl.ds(g * L, L)] = zi
            rv0[pl.ds(g * L, L)] = zi
            vv0[pl.ds(g * L, L)] = zf
        eb = ebase + NFULL * K
        pltpu.sync_copy(col_ref.at[pl.ds(eb, TAIL)], cv0.at[pl.ds(0, TAIL)])
        pltpu.sync_copy(row_ref.at[pl.ds(eb, TAIL)], rv0.at[pl.ds(0, TAIL)])
        pltpu.sync_copy(val_ref.at[pl.ds(eb, TAIL)], vv0.at[pl.ds(0, TAIL)])
        pltpu.async_copy(x_sp.at[cv0], rb0, semG[0]).wait()
        scale_chunk(0)
        pltpu.sync_copy(rb0, dst.at[rv0], add=True)

    def post_layer(rezero, scale):
        for sub in range(NSUB):
            r0l = row0 + sub * SUB
            r0g = cN + r0l
            pltpu.sync_copy(out_c.at[pl.ds(r0g, SUB)], tbuf)
            pltpu.sync_copy(xa.at[pl.ds(r0l, SUB)], t2)

            def eb(i, _):
                for k in range(HALF // L):
                    sl = pl.ds(k * L, L)
                    tbuf[i, sl] = (tbuf[i, sl] + t2[i, sl]) * scale
                return 0

            lax.fori_loop(0, SUB, eb, 0)
            pltpu.sync_copy(tbuf, out_c.at[pl.ds(r0g, SUB)])
        if rezero:
            zero_tbuf()
            zero_acc_stripe()

    for sub in range(NSUB):
        r0l = row0 + sub * SUB
        pltpu.sync_copy(poi.at[pl.ds(cN + r0l, SUB)], tbuf)
        pltpu.sync_copy(tbuf, xa.at[pl.ds(r0l, SUB)])
        pltpu.sync_copy(tbuf, out_c.at[pl.ds(cN + r0l, SUB)])
    zero_tbuf()
    zero_acc_stripe()
    plsc.subcore_barrier()

    for layer in range(2):
        spmm(tar_r, tar_c, tarv, xa, acc)
        plsc.subcore_barrier()
        spmm(src_r, src_c, srcv, acc, xa)
        plsc.subcore_barrier()
        post_layer(rezero=(layer == 0),
                   scale=jnp.float32(1.0 if layer == 0 else 1.0 / 3.0))
        plsc.subcore_barrier()


@jax.jit
def kernel(poi_embs, src_indices, src_values, tar_indices, tar_values):
    poi_cat = poi_embs.reshape(N, NC, HALF).transpose(1, 0, 2)
    poi_cat = jnp.pad(poi_cat, ((0, 0), (0, NP - N), (0, 0))).reshape(NC * NP, HALF)

    mesh = plsc.VectorSubcoreMesh(
        core_axis_name="c", subcore_axis_name="s", num_cores=NC, num_subcores=NS
    )
    run = pl.kernel(
        _body,
        out_type=[
            jax.ShapeDtypeStruct((NC * NP, HALF), _f32),
        ],
        mesh=mesh,
        compiler_params=pltpu.CompilerParams(use_tc_tiling_on_sc=False),
        scratch_types=[
            pltpu.VMEM_SHARED((NP, HALF), _f32),
            pltpu.VMEM_SHARED((NP, HALF), _f32),
        ]
        + [pltpu.VMEM((K,), _i32) for _ in range(2 * NB)]
        + [pltpu.VMEM((K,), _f32) for _ in range(NB)]
        + [pltpu.VMEM((K, HALF), _f32) for _ in range(NB)]
        + [
            pltpu.VMEM((SUB, HALF), _f32),
            pltpu.VMEM((SUB, HALF), _f32),
        ]
        + [pltpu.SemaphoreType.DMA for _ in range(3 * NB)],
    )
    (out_c,) = run(poi_cat, tar_indices[0], tar_indices[1], tar_values,
                   src_indices[0], src_indices[1], src_values)
    return out_c.reshape(NC, NP, HALF)[:, :N].transpose(1, 0, 2).reshape(N, D)
